# baseline (speedup 1.0000x reference)
import functools

import jax
import jax.numpy as jnp
from jax import lax
from jax.experimental import pallas as pl
from jax.experimental.pallas import tpu as pltpu
from jax.experimental.pallas import tpu_sc as plsc

VOCAB = 100000
MAXLEN = 200
EMBED = 128
BATCH = 1024

NC = 2
NS = 16
NW = NC * NS
TOK = BATCH * MAXLEN
LANES = 16
CHUNK = 64
CHUNKS = TOK // (NW * CHUNK)
NBUF = 10
AHEAD = 5
NSTEP = CHUNKS // NBUF
DO_ADD = True
DO_OUT = True


def _sc_kernel(x_hbm, tok_hbm, pos_hbm, out_hbm, idx_v, pos_v, rows_v, *sems):
    gs = sems[:NBUF]
    osm = sems[NBUF:]
    wid = lax.axis_index("s") * NC + lax.axis_index("c")
    base_c = wid * CHUNKS

    pltpu.sync_copy(x_hbm.at[wid], idx_v)
    pltpu.sync_copy(pos_hbm, pos_v)

    def gather_fire(c, p):
        pltpu.async_copy(tok_hbm.at[idx_v.at[c]], rows_v.at[p], gs[p])

    def gather_wait(c, p):
        pltpu.make_async_copy(tok_hbm.at[idx_v.at[c]],
                              rows_v.at[p], gs[p]).wait()

    def out_wait(p):
        pltpu.make_async_copy(rows_v.at[p],
                              out_hbm.at[pl.ds(0, CHUNK)], osm[p]).wait()

    for c0 in range(AHEAD):
        gather_fire(c0, c0)

    def body(t, carry):
        for k in range(NBUF):
            c = NBUF * t + k
            gather_wait(c, k)

            pbase = lax.rem(c * CHUNK, MAXLEN)

            if DO_ADD:
                @plsc.parallel_loop(0, CHUNK, step=1, unroll=4)
                def add_row(rr):
                    pr = lax.rem(pbase + rr, MAXLEN)
                    for j in range(EMBED // LANES):
                        plsc.addupdate(
                            rows_v.at[k, rr, pl.ds(j * LANES, LANES)],
                            pos_v[pr, pl.ds(j * LANES, LANES)])

            if DO_OUT:
                pltpu.async_copy(rows_v.at[k],
                                 out_hbm.at[pl.ds((base_c + c) * CHUNK, CHUNK)],
                                 osm[k])

            pn = (k + AHEAD) % NBUF
            if k < NBUF - AHEAD:
                if DO_OUT:
                    @pl.when(t >= 1)
                    def _():
                        out_wait(pn)

                gather_fire(c + AHEAD, pn)
            else:
                if DO_OUT:
                    @pl.when(t < NSTEP - 1)
                    def _():
                        out_wait(pn)
                        gather_fire(c + AHEAD, pn)
                else:
                    @pl.when(t < NSTEP - 1)
                    def _():
                        gather_fire(c + AHEAD, pn)
        return carry

    lax.fori_loop(0, NSTEP, body, 0)

    if DO_OUT:
        for p in range(NBUF):
            out_wait(p)


@jax.jit
def kernel(x, token_table, pos_table):
    x3 = x.astype(jnp.int32).reshape(NW, CHUNKS, CHUNK)
    mesh = plsc.VectorSubcoreMesh(core_axis_name="c", subcore_axis_name="s")
    k = functools.partial(
        pl.kernel,
        mesh=mesh,
        out_type=jax.ShapeDtypeStruct((TOK, EMBED), jnp.float32),
        scratch_types=[
            pltpu.VMEM((CHUNKS, CHUNK), jnp.int32),
            pltpu.VMEM((MAXLEN, EMBED), jnp.float32),
            pltpu.VMEM((NBUF, CHUNK, EMBED), jnp.float32),
        ] + [pltpu.SemaphoreType.DMA] * (2 * NBUF),
    )(_sc_kernel)
    out = k(x3, token_table, pos_table)
    return out.reshape(BATCH, MAXLEN, EMBED)


# speedup vs baseline: 1.0210x; 1.0210x over previous
import functools

import jax
import jax.numpy as jnp
from jax import lax
from jax.experimental import pallas as pl
from jax.experimental.pallas import tpu as pltpu
from jax.experimental.pallas import tpu_sc as plsc

VOCAB = 100000
MAXLEN = 200
EMBED = 128
BATCH = 1024

NC = 2
NS = 16
NW = NC * NS
TOK = BATCH * MAXLEN
LANES = 16
CHUNK = 128
CHUNKS = TOK // (NW * CHUNK)
NBUF = 5
AHEAD = 3
NSTEP = CHUNKS // NBUF
DO_ADD = True
DO_OUT = True


def _sc_kernel(x_hbm, tok_hbm, pos_hbm, out_hbm, idx_v, pos_v, rows_v, *sems):
    gs = sems[:NBUF]
    osm = sems[NBUF:]
    wid = lax.axis_index("s") * NC + lax.axis_index("c")
    base_c = wid * CHUNKS

    pltpu.sync_copy(x_hbm.at[wid], idx_v)
    pltpu.sync_copy(pos_hbm, pos_v)

    def gather_fire(c, p):
        pltpu.async_copy(tok_hbm.at[idx_v.at[c]], rows_v.at[p], gs[p])

    def gather_wait(c, p):
        pltpu.make_async_copy(tok_hbm.at[idx_v.at[c]],
                              rows_v.at[p], gs[p]).wait()

    def out_wait(p):
        pltpu.make_async_copy(rows_v.at[p],
                              out_hbm.at[pl.ds(0, CHUNK)], osm[p]).wait()

    for c0 in range(AHEAD):
        gather_fire(c0, c0)

    def body(t, carry):
        for k in range(NBUF):
            c = NBUF * t + k
            gather_wait(c, k)

            pbase = lax.rem(c * CHUNK, MAXLEN)

            if DO_ADD:
                @plsc.parallel_loop(0, CHUNK, step=1, unroll=4)
                def add_row(rr):
                    pr = lax.rem(pbase + rr, MAXLEN)
                    for j in range(EMBED // LANES):
                        plsc.addupdate(
                            rows_v.at[k, rr, pl.ds(j * LANES, LANES)],
                            pos_v[pr, pl.ds(j * LANES, LANES)])

            if DO_OUT:
                pltpu.async_copy(rows_v.at[k],
                                 out_hbm.at[pl.ds((base_c + c) * CHUNK, CHUNK)],
                                 osm[k])

            pn = (k + AHEAD) % NBUF
            if k < NBUF - AHEAD:
                if DO_OUT:
                    @pl.when(t >= 1)
                    def _():
                        out_wait(pn)

                gather_fire(c + AHEAD, pn)
            else:
                if DO_OUT:
                    @pl.when(t < NSTEP - 1)
                    def _():
                        out_wait(pn)
                        gather_fire(c + AHEAD, pn)
                else:
                    @pl.when(t < NSTEP - 1)
                    def _():
                        gather_fire(c + AHEAD, pn)
        return carry

    lax.fori_loop(0, NSTEP, body, 0)

    if DO_OUT:
        for p in range(NBUF):
            out_wait(p)


@jax.jit
def kernel(x, token_table, pos_table):
    x3 = x.astype(jnp.int32).reshape(NW, CHUNKS, CHUNK)
    mesh = plsc.VectorSubcoreMesh(core_axis_name="c", subcore_axis_name="s")
    k = functools.partial(
        pl.kernel,
        mesh=mesh,
        out_type=jax.ShapeDtypeStruct((TOK, EMBED), jnp.float32),
        scratch_types=[
            pltpu.VMEM((CHUNKS, CHUNK), jnp.int32),
            pltpu.VMEM((MAXLEN, EMBED), jnp.float32),
            pltpu.VMEM((NBUF, CHUNK, EMBED), jnp.float32),
        ] + [pltpu.SemaphoreType.DMA] * (2 * NBUF),
    )(_sc_kernel)
    out = k(x3, token_table, pos_table)
    return out.reshape(BATCH, MAXLEN, EMBED)
